# Initial kernel scaffold; baseline (speedup 1.0000x reference)
import functools

import jax
import jax.numpy as jnp
from jax import lax
from jax.experimental import pallas as pl
from jax.experimental.pallas import tpu as pltpu
from jax.experimental.pallas import tpu_sc as plsc

_N = 4096
_D = 64
_F = 128
_K = 20
_B = 8
_KP = 32
_BN_EPS = 1e-5
_ROWS = 512

_NC = 2
_NS = 16
_NW = _NC * _NS
_CHUNK = _N // _NW
_SUB = 16


def _topk_body(wb_ref, wa_ref, idx_ref):
    wb = wb_ref[...]
    wa = wa_ref[...]
    rs2 = jnp.sum(wb * wb, axis=1, keepdims=True)
    cs2 = jnp.sum(wa * wa, axis=1)
    cos = lax.dot_general(wb, wa, (((1,), (1,)), ((), ())),
                          preferred_element_type=jnp.float32)
    val = cos / (jnp.sqrt(rs2) * jnp.sqrt(cs2)[None, :])
    colio = lax.broadcasted_iota(jnp.int32, (_ROWS, _N), 1)
    sels = []
    for _ in range(_K):
        m = jnp.max(val, axis=1, keepdims=True)
        cand = jnp.where(val >= m, colio, jnp.int32(_N))
        sel = jnp.min(cand, axis=1, keepdims=True)
        sels.append(sel)
        val = jnp.where(colio == sel, -jnp.inf, val)
    r = pl.program_id(0)
    rowio = r * _ROWS + lax.broadcasted_iota(jnp.int32, (_ROWS, 1), 0)
    sels.extend([rowio] * (_KP - _K))
    idx_ref[...] = jnp.concatenate(sels, axis=1)


def _topk_call(emb_table):
    return pl.pallas_call(
        _topk_body,
        grid=(_N // _ROWS,),
        in_specs=[pl.BlockSpec((_ROWS, _D), lambda i: (i, 0)),
                  pl.BlockSpec((_N, _D), lambda i: (0, 0))],
        out_specs=pl.BlockSpec((_ROWS, _KP), lambda i: (i, 0)),
        out_shape=jax.ShapeDtypeStruct((_N, _KP), jnp.int32),
    )(emb_table, emb_table)


def _lin_body(x_ref, w_ref, emb_ref, ati_ref, atj_ref, aei_ref, aej_ref,
              xl_ref, ai_ref, aj_ref):
    x = x_ref[0]
    xl = jnp.dot(x, w_ref[...], preferred_element_type=jnp.float32)
    xl_ref[0] = xl
    emb = emb_ref[...]
    ei = jnp.sum(emb * aei_ref[...], axis=1)
    ej = jnp.sum(emb * aej_ref[...], axis=1)
    ai_ref[0, 0] = jnp.sum(xl * ati_ref[...], axis=1) + ei
    aj_ref[0, 0] = jnp.sum(xl * atj_ref[...], axis=1) + ej


def _lin_call(batch_tensor, lin_W, emb_table, att_i, att_j, att_em_i, att_em_j):
    vspec = pl.BlockSpec((1, _D), lambda b: (0, 0))
    return pl.pallas_call(
        _lin_body,
        grid=(_B,),
        in_specs=[pl.BlockSpec((1, _N, _F), lambda b: (b, 0, 0)),
                  pl.BlockSpec((_F, _D), lambda b: (0, 0)),
                  pl.BlockSpec((_N, _D), lambda b: (0, 0)),
                  vspec, vspec, vspec, vspec],
        out_specs=[pl.BlockSpec((1, _N, _D), lambda b: (b, 0, 0)),
                   pl.BlockSpec((1, 1, _N), lambda b: (b, 0, 0)),
                   pl.BlockSpec((1, 1, _N), lambda b: (b, 0, 0))],
        out_shape=[jax.ShapeDtypeStruct((_B, _N, _D), jnp.float32),
                   jax.ShapeDtypeStruct((_B, 1, _N), jnp.float32),
                   jax.ShapeDtypeStruct((_B, 1, _N), jnp.float32)],
    )(batch_tensor, lin_W, emb_table,
      att_i.reshape(1, _D), att_j.reshape(1, _D),
      att_em_i.reshape(1, _D), att_em_j.reshape(1, _D))


def _sc_body(xlin, aiH, ajH, tkH, out,
             tkf, ilist, ajv, aiv, rows, wmat, outbuf, sem):
    wid = lax.axis_index("s") * _NC + lax.axis_index("c")
    nbase = wid * _CHUNK
    pltpu.sync_copy(tkH.at[pl.ds(nbase * _KP, _CHUNK * _KP)], tkf)
    lane = lax.iota(jnp.int32, 16)

    def batch_body(b, c0):
        boff = b * _N
        pltpu.sync_copy(ajH.at[pl.ds(boff, _N)], ajv)
        pltpu.sync_copy(aiH.at[pl.ds(boff + nbase, _CHUNK)], aiv)

        def il_body(t, c1):
            o = t * 16
            ilist[pl.ds(o, 16)] = tkf[pl.ds(o, 16)] + boff
            return c1
        lax.fori_loop(0, _CHUNK * _KP // 16, il_body, 0)

        def sub_body(s, c2):
            sb = s * _SUB
            ib = sb * _KP
            cps = [pltpu.async_copy(
                       xlin.at[ilist.at[pl.ds(ib + j * 128, 128)]],
                       rows.at[pl.ds(j * 128, 128)], sem)
                   for j in range(_SUB * _KP // 128)]
            for cp in cps:
                cp.wait()
            g = nbase + sb + lane
            gbase = (sb + lane) * _KP
            ai16 = aiv[pl.ds(sb, 16)]
            a_self = ai16 + ajv[pl.ds(nbase + sb, 16)]
            a_self = jnp.maximum(a_self, 0.2 * a_self)
            alphas = []
            for k in range(_K):
                idxk = plsc.load_gather(tkf, [gbase + k])
                ajk = plsc.load_gather(ajv, [idxk])
                a = ai16 + ajk
                a = jnp.maximum(a, 0.2 * a)
                a = jnp.where(idxk == g, -1e9, a)
                alphas.append(a)
            m = a_self
            for a in alphas:
                m = jnp.maximum(m, a)
            e_self = jnp.exp(a_self - m)
            ssum = e_self
            es = []
            for a in alphas:
                e = jnp.exp(a - m)
                ssum = ssum + e
                es.append(e)
            d = ssum + 1e-16
            for k in range(_K):
                wmat[k, :] = es[k] / d
            wmat[_K, :] = e_self / d

            def node_body(i, c3):
                rb = i * _KP
                w0 = wmat[0, i]
                acc0 = w0 * rows[rb, pl.ds(0, 16)]
                acc1 = w0 * rows[rb, pl.ds(16, 16)]
                acc2 = w0 * rows[rb, pl.ds(32, 16)]
                acc3 = w0 * rows[rb, pl.ds(48, 16)]
                for k in range(1, _K + 1):
                    wk = wmat[k, i]
                    acc0 = acc0 + wk * rows[rb + k, pl.ds(0, 16)]
                    acc1 = acc1 + wk * rows[rb + k, pl.ds(16, 16)]
                    acc2 = acc2 + wk * rows[rb + k, pl.ds(32, 16)]
                    acc3 = acc3 + wk * rows[rb + k, pl.ds(48, 16)]
                outbuf[i, pl.ds(0, 16)] = acc0
                outbuf[i, pl.ds(16, 16)] = acc1
                outbuf[i, pl.ds(32, 16)] = acc2
                outbuf[i, pl.ds(48, 16)] = acc3
                return c3
            lax.fori_loop(0, _SUB, node_body, 0)
            pltpu.sync_copy(outbuf, out.at[pl.ds(boff + nbase + sb, _SUB)])
            return c2
        lax.fori_loop(0, _CHUNK // _SUB, sub_body, 0)
        return c0
    lax.fori_loop(0, _B, batch_body, 0)


def _sc_call(xlin_flat, ai_flat, aj_flat, tk_flat):
    f = pl.kernel(
        _sc_body,
        out_type=jax.ShapeDtypeStruct((_B * _N, _D), jnp.float32),
        mesh=plsc.VectorSubcoreMesh(core_axis_name="c", subcore_axis_name="s"),
        scratch_types=[
            pltpu.VMEM((_CHUNK * _KP,), jnp.int32),
            pltpu.VMEM((_CHUNK * _KP,), jnp.int32),
            pltpu.VMEM((_N,), jnp.float32),
            pltpu.VMEM((_CHUNK,), jnp.float32),
            pltpu.VMEM((_SUB * _KP, _D), jnp.float32),
            pltpu.VMEM((_K + 1, 16), jnp.float32),
            pltpu.VMEM((_SUB, _D), jnp.float32),
            pltpu.SemaphoreType.DMA,
        ],
    )
    return f(xlin_flat, ai_flat, aj_flat, tk_flat)


def _epi_body(agg_ref, emb_ref, glb_ref, g1_ref, b1_ref, g2_ref, b2_ref,
              wo_ref, o_ref):
    a = agg_ref[0] + glb_ref[...]
    h = jnp.maximum(a * g1_ref[...] + b1_ref[...], 0.0)
    o = jnp.maximum(h * emb_ref[...] * g2_ref[...] + b2_ref[...], 0.0)
    o_ref[0, 0] = jnp.sum(o * wo_ref[...], axis=1)


def _epi_call(agg, emb_table, glb, g1, b1, g2, b2, wo):
    vspec = pl.BlockSpec((1, _D), lambda b: (0, 0))
    return pl.pallas_call(
        _epi_body,
        grid=(_B,),
        in_specs=[pl.BlockSpec((1, _N, _D), lambda b: (b, 0, 0)),
                  pl.BlockSpec((_N, _D), lambda b: (0, 0)),
                  vspec, vspec, vspec, vspec, vspec, vspec],
        out_specs=pl.BlockSpec((1, 1, _N), lambda b: (b, 0, 0)),
        out_shape=jax.ShapeDtypeStruct((_B, 1, _N), jnp.float32),
    )(agg, emb_table, glb, g1, b1, g2, b2, wo)


def kernel(batch_tensor, org_edge_index, emb_table, lin_W, att_i, att_j,
           att_em_i, att_em_j, gl_bias, bn1_gamma, bn1_beta, bn_out_gamma,
           bn_out_beta, out_W, out_b):
    del org_edge_index
    tk32 = _topk_call(emb_table)
    xlin, ai, aj = _lin_call(batch_tensor, lin_W, emb_table,
                             att_i, att_j, att_em_i, att_em_j)
    agg = _sc_call(xlin.reshape(_B * _N, _D), ai.reshape(-1), aj.reshape(-1),
                   tk32.reshape(-1))
    c = (1.0 + _BN_EPS) ** -0.5
    o = _epi_call(agg.reshape(_B, _N, _D), emb_table,
                  gl_bias.reshape(1, _D),
                  (bn1_gamma * c).reshape(1, _D), bn1_beta.reshape(1, _D),
                  (bn_out_gamma * c).reshape(1, _D), bn_out_beta.reshape(1, _D),
                  out_W.reshape(1, _D))
    return o.reshape(_B, _N) + out_b[0]


# speedup vs baseline: 38.8280x; 38.8280x over previous
import functools

import jax
import jax.numpy as jnp
from jax import lax
from jax.experimental import pallas as pl
from jax.experimental.pallas import tpu as pltpu
from jax.experimental.pallas import tpu_sc as plsc

_N = 4096
_D = 64
_F = 128
_K = 20
_B = 8
_KP = 32
_BN_EPS = 1e-5
_ROWS = 512

_NC = 2
_NS = 16
_NW = _NC * _NS
_CHUNK = _N // _NW
_SUB = 16


def _topk_body(wb_ref, wa_ref, idx_ref):
    wb = wb_ref[...]
    wa = wa_ref[...]
    rs2 = jnp.sum(wb * wb, axis=1, keepdims=True)
    cs2 = jnp.sum(wa * wa, axis=1)
    cos = lax.dot_general(wb, wa, (((1,), (1,)), ((), ())),
                          preferred_element_type=jnp.float32)
    val = cos / (jnp.sqrt(rs2) * jnp.sqrt(cs2)[None, :])
    colio = lax.broadcasted_iota(jnp.int32, (_ROWS, _N), 1)
    sels = []
    for _ in range(_K):
        m = jnp.max(val, axis=1, keepdims=True)
        cand = jnp.where(val >= m, colio, jnp.int32(_N))
        sel = jnp.min(cand, axis=1, keepdims=True)
        sels.append(sel)
        val = jnp.where(colio == sel, -jnp.inf, val)
    r = pl.program_id(0)
    rowio = r * _ROWS + lax.broadcasted_iota(jnp.int32, (_ROWS, 1), 0)
    sels.extend([rowio] * (_KP - _K))
    idx_ref[...] = jnp.concatenate(sels, axis=1)


def _topk_call(emb_table):
    return pl.pallas_call(
        _topk_body,
        grid=(_N // _ROWS,),
        in_specs=[pl.BlockSpec((_ROWS, _D), lambda i: (i, 0)),
                  pl.BlockSpec((_N, _D), lambda i: (0, 0))],
        out_specs=pl.BlockSpec((_ROWS, _KP), lambda i: (i, 0)),
        out_shape=jax.ShapeDtypeStruct((_N, _KP), jnp.int32),
    )(emb_table, emb_table)


def _lin_body(x_ref, w_ref, emb_ref, ati_ref, atj_ref, aei_ref, aej_ref,
              xl_ref, ai_ref, aj_ref):
    x = x_ref[0]
    xl = jnp.dot(x, w_ref[...], preferred_element_type=jnp.float32)
    xl_ref[0] = xl
    emb = emb_ref[...]
    ei = jnp.sum(emb * aei_ref[...], axis=1)
    ej = jnp.sum(emb * aej_ref[...], axis=1)
    ai_ref[0, 0] = jnp.sum(xl * ati_ref[...], axis=1) + ei
    aj_ref[0, 0] = jnp.sum(xl * atj_ref[...], axis=1) + ej


def _lin_call(batch_tensor, lin_W, emb_table, att_i, att_j, att_em_i, att_em_j):
    vspec = pl.BlockSpec((1, _D), lambda b: (0, 0))
    return pl.pallas_call(
        _lin_body,
        grid=(_B,),
        in_specs=[pl.BlockSpec((1, _N, _F), lambda b: (b, 0, 0)),
                  pl.BlockSpec((_F, _D), lambda b: (0, 0)),
                  pl.BlockSpec((_N, _D), lambda b: (0, 0)),
                  vspec, vspec, vspec, vspec],
        out_specs=[pl.BlockSpec((1, _N, _D), lambda b: (b, 0, 0)),
                   pl.BlockSpec((1, 1, _N), lambda b: (b, 0, 0)),
                   pl.BlockSpec((1, 1, _N), lambda b: (b, 0, 0))],
        out_shape=[jax.ShapeDtypeStruct((_B, _N, _D), jnp.float32),
                   jax.ShapeDtypeStruct((_B, 1, _N), jnp.float32),
                   jax.ShapeDtypeStruct((_B, 1, _N), jnp.float32)],
    )(batch_tensor, lin_W, emb_table,
      att_i.reshape(1, _D), att_j.reshape(1, _D),
      att_em_i.reshape(1, _D), att_em_j.reshape(1, _D))


def _sc_body(xlin, aiH, ajH, tkH, out,
             tkf, ilist, ajv, aiv, rows, wmat, outbuf, sem):
    wid = lax.axis_index("s") * _NC + lax.axis_index("c")
    nbase = wid * _CHUNK
    pltpu.sync_copy(tkH.at[pl.ds(nbase * _KP, _CHUNK * _KP)], tkf)
    lane = lax.iota(jnp.int32, 16)

    def batch_body(b, c0):
        boff = b * _N
        pltpu.sync_copy(ajH.at[pl.ds(boff, _N)], ajv)
        pltpu.sync_copy(aiH.at[pl.ds(boff + nbase, _CHUNK)], aiv)

        def il_body(t, c1):
            o = t * 16
            ilist[pl.ds(o, 16)] = tkf[pl.ds(o, 16)] + boff
            return c1
        lax.fori_loop(0, _CHUNK * _KP // 16, il_body, 0)

        def sub_body(s, c2):
            sb = s * _SUB
            ib = sb * _KP
            cps = [pltpu.async_copy(
                       xlin.at[ilist.at[pl.ds(ib + j * 128, 128)]],
                       rows.at[pl.ds(j * 128, 128)], sem)
                   for j in range(_SUB * _KP // 128)]
            for cp in cps:
                cp.wait()
            g = nbase + sb + lane
            gbase = (sb + lane) * _KP
            ai16 = aiv[pl.ds(sb, 16)]
            a_self = ai16 + ajv[pl.ds(nbase + sb, 16)]
            a_self = jnp.maximum(a_self, 0.2 * a_self)
            alphas = []
            for k in range(_K):
                idxk = plsc.load_gather(tkf, [gbase + k])
                ajk = plsc.load_gather(ajv, [idxk])
                a = ai16 + ajk
                a = jnp.maximum(a, 0.2 * a)
                a = jnp.where(idxk == g, -1e9, a)
                alphas.append(a)
            m = a_self
            for a in alphas:
                m = jnp.maximum(m, a)
            e_self = jnp.exp(a_self - m)
            ssum = e_self
            es = []
            for a in alphas:
                e = jnp.exp(a - m)
                ssum = ssum + e
                es.append(e)
            d = ssum + 1e-16
            for k in range(_K):
                plsc.store_scatter(wmat, [lane, jnp.zeros((16,), jnp.int32) + k],
                                   es[k] / d)
            plsc.store_scatter(wmat, [lane, jnp.zeros((16,), jnp.int32) + _K],
                               e_self / d)

            def node_body(i, c3):
                rb = i * _KP
                wlo = wmat[i, pl.ds(0, 16)]
                whi = wmat[i, pl.ds(16, 16)]
                w0 = wlo[0]
                acc0 = w0 * rows[rb, pl.ds(0, 16)]
                acc1 = w0 * rows[rb, pl.ds(16, 16)]
                acc2 = w0 * rows[rb, pl.ds(32, 16)]
                acc3 = w0 * rows[rb, pl.ds(48, 16)]
                for k in range(1, _K + 1):
                    wk = wlo[k] if k < 16 else whi[k - 16]
                    acc0 = acc0 + wk * rows[rb + k, pl.ds(0, 16)]
                    acc1 = acc1 + wk * rows[rb + k, pl.ds(16, 16)]
                    acc2 = acc2 + wk * rows[rb + k, pl.ds(32, 16)]
                    acc3 = acc3 + wk * rows[rb + k, pl.ds(48, 16)]
                outbuf[i, pl.ds(0, 16)] = acc0
                outbuf[i, pl.ds(16, 16)] = acc1
                outbuf[i, pl.ds(32, 16)] = acc2
                outbuf[i, pl.ds(48, 16)] = acc3
                return c3
            lax.fori_loop(0, _SUB, node_body, 0)
            pltpu.sync_copy(outbuf, out.at[pl.ds(boff + nbase + sb, _SUB)])
            return c2
        lax.fori_loop(0, _CHUNK // _SUB, sub_body, 0)
        return c0
    lax.fori_loop(0, _B, batch_body, 0)


def _sc_call(xlin_flat, ai_flat, aj_flat, tk_flat):
    f = pl.kernel(
        _sc_body,
        out_type=jax.ShapeDtypeStruct((_B * _N, _D), jnp.float32),
        mesh=plsc.VectorSubcoreMesh(core_axis_name="c", subcore_axis_name="s",
                                    num_cores=_NC, num_subcores=_NS),
        compiler_params=pltpu.CompilerParams(needs_layout_passes=False,
                                             use_tc_tiling_on_sc=False),
        scratch_types=[
            pltpu.VMEM((_CHUNK * _KP,), jnp.int32),
            pltpu.VMEM((_CHUNK * _KP,), jnp.int32),
            pltpu.VMEM((_N,), jnp.float32),
            pltpu.VMEM((_CHUNK,), jnp.float32),
            pltpu.VMEM((_SUB * _KP, _D), jnp.float32),
            pltpu.VMEM((_SUB, _KP), jnp.float32),
            pltpu.VMEM((_SUB, _D), jnp.float32),
            pltpu.SemaphoreType.DMA,
        ],
    )
    return f(xlin_flat, ai_flat, aj_flat, tk_flat)


def _epi_body(agg_ref, emb_ref, glb_ref, g1_ref, b1_ref, g2_ref, b2_ref,
              wo_ref, o_ref):
    a = agg_ref[0] + glb_ref[...]
    h = jnp.maximum(a * g1_ref[...] + b1_ref[...], 0.0)
    o = jnp.maximum(h * emb_ref[...] * g2_ref[...] + b2_ref[...], 0.0)
    o_ref[0, 0] = jnp.sum(o * wo_ref[...], axis=1)


def _epi_call(agg, emb_table, glb, g1, b1, g2, b2, wo):
    vspec = pl.BlockSpec((1, _D), lambda b: (0, 0))
    return pl.pallas_call(
        _epi_body,
        grid=(_B,),
        in_specs=[pl.BlockSpec((1, _N, _D), lambda b: (b, 0, 0)),
                  pl.BlockSpec((_N, _D), lambda b: (0, 0)),
                  vspec, vspec, vspec, vspec, vspec, vspec],
        out_specs=pl.BlockSpec((1, 1, _N), lambda b: (b, 0, 0)),
        out_shape=jax.ShapeDtypeStruct((_B, 1, _N), jnp.float32),
    )(agg, emb_table, glb, g1, b1, g2, b2, wo)


def kernel(batch_tensor, org_edge_index, emb_table, lin_W, att_i, att_j,
           att_em_i, att_em_j, gl_bias, bn1_gamma, bn1_beta, bn_out_gamma,
           bn_out_beta, out_W, out_b):
    del org_edge_index
    tk32 = _topk_call(emb_table)
    xlin, ai, aj = _lin_call(batch_tensor, lin_W, emb_table,
                             att_i, att_j, att_em_i, att_em_j)
    agg = _sc_call(xlin.reshape(_B * _N, _D), ai.reshape(-1), aj.reshape(-1),
                   tk32.reshape(-1))
    c = (1.0 + _BN_EPS) ** -0.5
    o = _epi_call(agg.reshape(_B, _N, _D), emb_table,
                  gl_bias.reshape(1, _D),
                  (bn1_gamma * c).reshape(1, _D), bn1_beta.reshape(1, _D),
                  (bn_out_gamma * c).reshape(1, _D), bn_out_beta.reshape(1, _D),
                  out_W.reshape(1, _D))
    return o.reshape(_B, _N) + out_b[0]


# speedup vs baseline: 46.7960x; 1.2052x over previous
import functools

import jax
import jax.numpy as jnp
from jax import lax
from jax.experimental import pallas as pl
from jax.experimental.pallas import tpu as pltpu
from jax.experimental.pallas import tpu_sc as plsc

_N = 4096
_D = 64
_F = 128
_K = 20
_B = 8
_KP = 32
_KL = 24
_BN_EPS = 1e-5
_ROWS = 512

_NC = 2
_NS = 16
_NW = _NC * _NS
_CHUNK = _N // _NW
_SUB = 16


def _topk_body(wb_ref, wa_ref, idx_ref):
    wb = wb_ref[...]
    wa = wa_ref[...]
    cs2 = jnp.sum(wa * wa, axis=1)
    cos = lax.dot_general(wb, wa, (((1,), (1,)), ((), ())),
                          preferred_element_type=jnp.float32)
    val = cos / jnp.sqrt(cs2)[None, :]
    colio = lax.broadcasted_iota(jnp.int32, (_ROWS, _N), 1)
    sels = []
    sel = None
    for _ in range(_K):
        if sel is not None:
            val = jnp.where(colio == sel, -jnp.inf, val)
        m = jnp.max(val, axis=1, keepdims=True)
        sel = jnp.min(jnp.where(val < m, jnp.int32(_N), colio),
                      axis=1, keepdims=True)
        sels.append(sel)
    r = pl.program_id(0)
    rowio = r * _ROWS + lax.broadcasted_iota(jnp.int32, (_ROWS, 1), 0)
    sels.extend([rowio] * (_KP - _K))
    idx_ref[...] = jnp.concatenate(sels, axis=1)


def _topk_call(emb_table):
    return pl.pallas_call(
        _topk_body,
        grid=(_N // _ROWS,),
        in_specs=[pl.BlockSpec((_ROWS, _D), lambda i: (i, 0)),
                  pl.BlockSpec((_N, _D), lambda i: (0, 0))],
        out_specs=pl.BlockSpec((_ROWS, _KP), lambda i: (i, 0)),
        out_shape=jax.ShapeDtypeStruct((_N, _KP), jnp.int32),
    )(emb_table, emb_table)


def _lin_body(x_ref, w_ref, emb_ref, ati_ref, atj_ref, aei_ref, aej_ref,
              xl_ref, ai_ref, aj_ref):
    x = x_ref[0]
    xl = jnp.dot(x, w_ref[...], preferred_element_type=jnp.float32)
    xl_ref[0] = xl
    emb = emb_ref[...]
    ei = jnp.sum(emb * aei_ref[...], axis=1)
    ej = jnp.sum(emb * aej_ref[...], axis=1)
    ai_ref[0, 0] = jnp.sum(xl * ati_ref[...], axis=1) + ei
    aj_ref[0, 0] = jnp.sum(xl * atj_ref[...], axis=1) + ej


def _lin_call(batch_tensor, lin_W, emb_table, att_i, att_j, att_em_i, att_em_j):
    vspec = pl.BlockSpec((1, _D), lambda b: (0, 0))
    return pl.pallas_call(
        _lin_body,
        grid=(_B,),
        in_specs=[pl.BlockSpec((1, _N, _F), lambda b: (b, 0, 0)),
                  pl.BlockSpec((_F, _D), lambda b: (0, 0)),
                  pl.BlockSpec((_N, _D), lambda b: (0, 0)),
                  vspec, vspec, vspec, vspec],
        out_specs=[pl.BlockSpec((1, _N, _D), lambda b: (b, 0, 0)),
                   pl.BlockSpec((1, 1, _N), lambda b: (b, 0, 0)),
                   pl.BlockSpec((1, 1, _N), lambda b: (b, 0, 0))],
        out_shape=[jax.ShapeDtypeStruct((_B, _N, _D), jnp.float32),
                   jax.ShapeDtypeStruct((_B, 1, _N), jnp.float32),
                   jax.ShapeDtypeStruct((_B, 1, _N), jnp.float32)],
    )(batch_tensor, lin_W, emb_table,
      att_i.reshape(1, _D), att_j.reshape(1, _D),
      att_em_i.reshape(1, _D), att_em_j.reshape(1, _D))


def _sc_body(xlin, aiH, ajH, tkH, out,
             tkf, ilist, ajv, aiv, rows, wmat, outbuf, sem, osem):
    wid = lax.axis_index("s") * _NC + lax.axis_index("c")
    nbase = wid * _CHUNK
    pltpu.sync_copy(tkH.at[pl.ds(nbase * _KP, _CHUNK * _KP)], tkf)
    lane = lax.iota(jnp.int32, 16)
    nsub = _CHUNK // _SUB
    ndma = _SUB * _KL // 128

    def fire(s):
        buf = lax.rem(s, 2)
        ib = s * _SUB * _KL
        for j in range(ndma):
            pltpu.async_copy(
                xlin.at[ilist.at[pl.ds(ib + j * 128, 128)]],
                rows.at[buf, pl.ds(j * 128, 128)], sem.at[buf])

    def drain(s):
        buf = lax.rem(s, 2)
        ib = s * _SUB * _KL
        for j in range(ndma):
            pltpu.make_async_copy(
                xlin.at[ilist.at[pl.ds(ib + j * 128, 128)]],
                rows.at[buf, pl.ds(j * 128, 128)], sem.at[buf]).wait()

    def batch_body(b, c0):
        boff = b * _N
        pltpu.sync_copy(ajH.at[pl.ds(boff, _N)], ajv)
        pltpu.sync_copy(aiH.at[pl.ds(boff + nbase, _CHUNK)], aiv)

        def il_body(i, c1):
            r0 = tkf[pl.ds(i * _KP, 16)] + boff
            ilist[pl.ds(i * _KL, 16)] = r0
            r1 = tkf[pl.ds(i * _KP + 8, 16)] + boff
            ilist[pl.ds(i * _KL + 8, 16)] = r1
            return c1
        lax.fori_loop(0, _CHUNK, il_body, 0)
        fire(0)

        def sub_body(s, c2):
            sb = s * _SUB
            buf = lax.rem(s, 2)

            @pl.when(s < nsub - 1)
            def _():
                fire(s + 1)
            drain(s)

            @pl.when(s >= 2)
            def _():
                pltpu.make_async_copy(
                    outbuf.at[buf],
                    out.at[pl.ds(boff + nbase + (s - 2) * _SUB, _SUB)],
                    osem.at[buf]).wait()
            g = nbase + sb + lane
            gbase = (sb + lane) * _KP
            ai16 = aiv[pl.ds(sb, 16)]
            a_self = ai16 + ajv[pl.ds(nbase + sb, 16)]
            a_self = jnp.maximum(a_self, 0.2 * a_self)
            alphas = []
            for k in range(_K):
                idxk = plsc.load_gather(tkf, [gbase + k])
                ajk = plsc.load_gather(ajv, [idxk])
                a = ai16 + ajk
                a = jnp.maximum(a, 0.2 * a)
                a = jnp.where(idxk == g, -1e9, a)
                alphas.append(a)
            m = a_self
            for a in alphas:
                m = jnp.maximum(m, a)
            e_self = jnp.exp(a_self - m)
            ssum = e_self
            es = []
            for a in alphas:
                e = jnp.exp(a - m)
                ssum = ssum + e
                es.append(e)
            d = ssum + 1e-16
            for k in range(_K):
                plsc.store_scatter(wmat, [lane, jnp.zeros((16,), jnp.int32) + k],
                                   es[k] / d)
            plsc.store_scatter(wmat, [lane, jnp.zeros((16,), jnp.int32) + _K],
                               e_self / d)

            def node_body(i, c3):
                rb = i * _KL
                wlo = wmat[i, pl.ds(0, 16)]
                whi = wmat[i, pl.ds(16, 16)]
                w0 = wlo[0]
                acc0 = w0 * rows[buf, rb, pl.ds(0, 16)]
                acc1 = w0 * rows[buf, rb, pl.ds(16, 16)]
                acc2 = w0 * rows[buf, rb, pl.ds(32, 16)]
                acc3 = w0 * rows[buf, rb, pl.ds(48, 16)]
                for k in range(1, _K + 1):
                    wk = wlo[k] if k < 16 else whi[k - 16]
                    acc0 = acc0 + wk * rows[buf, rb + k, pl.ds(0, 16)]
                    acc1 = acc1 + wk * rows[buf, rb + k, pl.ds(16, 16)]
                    acc2 = acc2 + wk * rows[buf, rb + k, pl.ds(32, 16)]
                    acc3 = acc3 + wk * rows[buf, rb + k, pl.ds(48, 16)]
                outbuf[buf, i, pl.ds(0, 16)] = acc0
                outbuf[buf, i, pl.ds(16, 16)] = acc1
                outbuf[buf, i, pl.ds(32, 16)] = acc2
                outbuf[buf, i, pl.ds(48, 16)] = acc3
                return c3
            lax.fori_loop(0, _SUB, node_body, 0)
            pltpu.async_copy(outbuf.at[buf],
                             out.at[pl.ds(boff + nbase + sb, _SUB)],
                             osem.at[buf])
            return c2
        lax.fori_loop(0, nsub, sub_body, 0)
        for t in (nsub - 2, nsub - 1):
            pltpu.make_async_copy(
                outbuf.at[t % 2],
                out.at[pl.ds(boff + nbase + t * _SUB, _SUB)],
                osem.at[t % 2]).wait()
        return c0
    lax.fori_loop(0, _B, batch_body, 0)


def _sc_call(xlin_flat, ai_flat, aj_flat, tk_flat):
    f = pl.kernel(
        _sc_body,
        out_type=jax.ShapeDtypeStruct((_B * _N, _D), jnp.float32),
        mesh=plsc.VectorSubcoreMesh(core_axis_name="c", subcore_axis_name="s",
                                    num_cores=_NC, num_subcores=_NS),
        compiler_params=pltpu.CompilerParams(needs_layout_passes=False,
                                             use_tc_tiling_on_sc=False),
        scratch_types=[
            pltpu.VMEM((_CHUNK * _KP,), jnp.int32),
            pltpu.VMEM((_CHUNK * _KL,), jnp.int32),
            pltpu.VMEM((_N,), jnp.float32),
            pltpu.VMEM((_CHUNK,), jnp.float32),
            pltpu.VMEM((2, _SUB * _KL, _D), jnp.float32),
            pltpu.VMEM((_SUB, _KP), jnp.float32),
            pltpu.VMEM((2, _SUB, _D), jnp.float32),
            pltpu.SemaphoreType.DMA((2,)),
            pltpu.SemaphoreType.DMA((2,)),
        ],
    )
    return f(xlin_flat, ai_flat, aj_flat, tk_flat)


def _epi_body(agg_ref, emb_ref, glb_ref, g1_ref, b1_ref, g2_ref, b2_ref,
              wo_ref, o_ref):
    a = agg_ref[0] + glb_ref[...]
    h = jnp.maximum(a * g1_ref[...] + b1_ref[...], 0.0)
    o = jnp.maximum(h * emb_ref[...] * g2_ref[...] + b2_ref[...], 0.0)
    o_ref[0, 0] = jnp.sum(o * wo_ref[...], axis=1)


def _epi_call(agg, emb_table, glb, g1, b1, g2, b2, wo):
    vspec = pl.BlockSpec((1, _D), lambda b: (0, 0))
    return pl.pallas_call(
        _epi_body,
        grid=(_B,),
        in_specs=[pl.BlockSpec((1, _N, _D), lambda b: (b, 0, 0)),
                  pl.BlockSpec((_N, _D), lambda b: (0, 0)),
                  vspec, vspec, vspec, vspec, vspec, vspec],
        out_specs=pl.BlockSpec((1, 1, _N), lambda b: (b, 0, 0)),
        out_shape=jax.ShapeDtypeStruct((_B, 1, _N), jnp.float32),
    )(agg, emb_table, glb, g1, b1, g2, b2, wo)


def kernel(batch_tensor, org_edge_index, emb_table, lin_W, att_i, att_j,
           att_em_i, att_em_j, gl_bias, bn1_gamma, bn1_beta, bn_out_gamma,
           bn_out_beta, out_W, out_b):
    del org_edge_index
    tk32 = _topk_call(emb_table)
    xlin, ai, aj = _lin_call(batch_tensor, lin_W, emb_table,
                             att_i, att_j, att_em_i, att_em_j)
    agg = _sc_call(xlin.reshape(_B * _N, _D), ai.reshape(-1), aj.reshape(-1),
                   tk32.reshape(-1))
    c = (1.0 + _BN_EPS) ** -0.5
    o = _epi_call(agg.reshape(_B, _N, _D), emb_table,
                  gl_bias.reshape(1, _D),
                  (bn1_gamma * c).reshape(1, _D), bn1_beta.reshape(1, _D),
                  (bn_out_gamma * c).reshape(1, _D), bn_out_beta.reshape(1, _D),
                  out_W.reshape(1, _D))
    return o.reshape(_B, _N) + out_b[0]


# speedup vs baseline: 52.4818x; 1.1215x over previous
import functools

import jax
import jax.numpy as jnp
from jax import lax
from jax.experimental import pallas as pl
from jax.experimental.pallas import tpu as pltpu
from jax.experimental.pallas import tpu_sc as plsc

_N = 4096
_D = 64
_F = 128
_K = 20
_B = 8
_KP = 32
_KL = 24
_BN_EPS = 1e-5
_ROWS = 256

_NC = 2
_NS = 16
_NW = _NC * _NS
_CHUNK = _N // _NW
_SUB = 16


def _topk_body(wb_ref, wa_ref, idx_ref):
    wb = wb_ref[...]
    wa = wa_ref[...]
    cs2 = jnp.sum(wa * wa, axis=1)
    cos = lax.dot_general(wb, wa, (((1,), (1,)), ((), ())),
                          preferred_element_type=jnp.float32)
    val = cos / jnp.sqrt(cs2)[None, :]
    colio = lax.broadcasted_iota(jnp.int32, (_ROWS, _N), 1)
    sels = []
    sel = None
    for _ in range(_K):
        if sel is not None:
            val = jnp.where(colio == sel, -jnp.inf, val)
        m = jnp.max(val, axis=1, keepdims=True)
        sel = jnp.min(jnp.where(val < m, jnp.int32(_N), colio),
                      axis=1, keepdims=True)
        sels.append(sel)
    r = pl.program_id(0)
    rowio = r * _ROWS + lax.broadcasted_iota(jnp.int32, (_ROWS, 1), 0)
    sels.extend([rowio] * (_KP - _K))
    idx_ref[...] = jnp.concatenate(sels, axis=1)


def _topk_call(emb_table):
    return pl.pallas_call(
        _topk_body,
        grid=(_N // _ROWS,),
        in_specs=[pl.BlockSpec((_ROWS, _D), lambda i: (i, 0)),
                  pl.BlockSpec((_N, _D), lambda i: (0, 0))],
        out_specs=pl.BlockSpec((_ROWS, _KP), lambda i: (i, 0)),
        out_shape=jax.ShapeDtypeStruct((_N, _KP), jnp.int32),
    )(emb_table, emb_table)


def _lin_body(x_ref, w_ref, emb_ref, ati_ref, atj_ref, aei_ref, aej_ref,
              xl_ref, ai_ref, aj_ref):
    x = x_ref[0]
    xl = jnp.dot(x, w_ref[...], preferred_element_type=jnp.float32)
    xl_ref[0] = xl
    emb = emb_ref[...]
    ei = jnp.sum(emb * aei_ref[...], axis=1)
    ej = jnp.sum(emb * aej_ref[...], axis=1)
    ai_ref[0, 0] = jnp.sum(xl * ati_ref[...], axis=1) + ei
    aj_ref[0, 0] = jnp.sum(xl * atj_ref[...], axis=1) + ej


def _lin_call(batch_tensor, lin_W, emb_table, att_i, att_j, att_em_i, att_em_j):
    vspec = pl.BlockSpec((1, _D), lambda b: (0, 0))
    return pl.pallas_call(
        _lin_body,
        grid=(_B,),
        in_specs=[pl.BlockSpec((1, _N, _F), lambda b: (b, 0, 0)),
                  pl.BlockSpec((_F, _D), lambda b: (0, 0)),
                  pl.BlockSpec((_N, _D), lambda b: (0, 0)),
                  vspec, vspec, vspec, vspec],
        out_specs=[pl.BlockSpec((1, _N, _D), lambda b: (b, 0, 0)),
                   pl.BlockSpec((1, 1, _N), lambda b: (b, 0, 0)),
                   pl.BlockSpec((1, 1, _N), lambda b: (b, 0, 0))],
        out_shape=[jax.ShapeDtypeStruct((_B, _N, _D), jnp.float32),
                   jax.ShapeDtypeStruct((_B, 1, _N), jnp.float32),
                   jax.ShapeDtypeStruct((_B, 1, _N), jnp.float32)],
    )(batch_tensor, lin_W, emb_table,
      att_i.reshape(1, _D), att_j.reshape(1, _D),
      att_em_i.reshape(1, _D), att_em_j.reshape(1, _D))


def _sc_body(xlin, aiH, ajH, tkH, out,
             tkf, ilist, ajv, aiv, rows, wmat, outbuf, sem, osem):
    wid = lax.axis_index("s") * _NC + lax.axis_index("c")
    nbase = wid * _CHUNK
    pltpu.sync_copy(tkH.at[pl.ds(nbase * _KP, _CHUNK * _KP)], tkf)
    lane = lax.iota(jnp.int32, 16)
    nsub = _CHUNK // _SUB
    ndma = _SUB * _KL // 128

    def fire(s):
        buf = lax.rem(s, 2)
        ib = s * _SUB * _KL
        for j in range(ndma):
            pltpu.async_copy(
                xlin.at[ilist.at[pl.ds(ib + j * 128, 128)]],
                rows.at[buf, pl.ds(j * 128, 128)], sem.at[buf])

    def drain(s):
        buf = lax.rem(s, 2)
        ib = s * _SUB * _KL
        for j in range(ndma):
            pltpu.make_async_copy(
                xlin.at[ilist.at[pl.ds(ib + j * 128, 128)]],
                rows.at[buf, pl.ds(j * 128, 128)], sem.at[buf]).wait()

    def batch_body(b, c0):
        boff = b * _N
        pltpu.sync_copy(ajH.at[pl.ds(boff, _N)], ajv)
        pltpu.sync_copy(aiH.at[pl.ds(boff + nbase, _CHUNK)], aiv)

        def il_body(i, c1):
            r0 = tkf[pl.ds(i * _KP, 16)] + boff
            ilist[pl.ds(i * _KL, 16)] = r0
            r1 = tkf[pl.ds(i * _KP + 8, 16)] + boff
            ilist[pl.ds(i * _KL + 8, 16)] = r1
            return c1
        lax.fori_loop(0, _CHUNK, il_body, 0)
        fire(0)

        def sub_body(s, c2):
            sb = s * _SUB
            buf = lax.rem(s, 2)

            @pl.when(s < nsub - 1)
            def _():
                fire(s + 1)
            drain(s)

            @pl.when(s >= 2)
            def _():
                pltpu.make_async_copy(
                    outbuf.at[buf],
                    out.at[pl.ds(boff + nbase + (s - 2) * _SUB, _SUB)],
                    osem.at[buf]).wait()
            g = nbase + sb + lane
            gbase = (sb + lane) * _KP
            ai16 = aiv[pl.ds(sb, 16)]
            a_self = ai16 + ajv[pl.ds(nbase + sb, 16)]
            a_self = jnp.maximum(a_self, 0.2 * a_self)
            alphas = []
            for k in range(_K):
                idxk = plsc.load_gather(tkf, [gbase + k])
                ajk = plsc.load_gather(ajv, [idxk])
                a = ai16 + ajk
                a = jnp.maximum(a, 0.2 * a)
                a = jnp.where(idxk == g, -1e9, a)
                alphas.append(a)
            m = a_self
            for a in alphas:
                m = jnp.maximum(m, a)
            e_self = jnp.exp(a_self - m)
            ssum = e_self
            es = []
            for a in alphas:
                e = jnp.exp(a - m)
                ssum = ssum + e
                es.append(e)
            d = ssum + 1e-16
            for k in range(_K):
                plsc.store_scatter(wmat, [lane, jnp.zeros((16,), jnp.int32) + k],
                                   es[k] / d)
            plsc.store_scatter(wmat, [lane, jnp.zeros((16,), jnp.int32) + _K],
                               e_self / d)

            def node_body(i, c3):
                rb = i * _KL
                wlo = wmat[i, pl.ds(0, 16)]
                whi = wmat[i, pl.ds(16, 16)]
                w0 = wlo[0]
                acc0 = w0 * rows[buf, rb, pl.ds(0, 16)]
                acc1 = w0 * rows[buf, rb, pl.ds(16, 16)]
                acc2 = w0 * rows[buf, rb, pl.ds(32, 16)]
                acc3 = w0 * rows[buf, rb, pl.ds(48, 16)]
                for k in range(1, _K + 1):
                    wk = wlo[k] if k < 16 else whi[k - 16]
                    acc0 = acc0 + wk * rows[buf, rb + k, pl.ds(0, 16)]
                    acc1 = acc1 + wk * rows[buf, rb + k, pl.ds(16, 16)]
                    acc2 = acc2 + wk * rows[buf, rb + k, pl.ds(32, 16)]
                    acc3 = acc3 + wk * rows[buf, rb + k, pl.ds(48, 16)]
                outbuf[buf, i, pl.ds(0, 16)] = acc0
                outbuf[buf, i, pl.ds(16, 16)] = acc1
                outbuf[buf, i, pl.ds(32, 16)] = acc2
                outbuf[buf, i, pl.ds(48, 16)] = acc3
                return c3
            lax.fori_loop(0, _SUB, node_body, 0)
            pltpu.async_copy(outbuf.at[buf],
                             out.at[pl.ds(boff + nbase + sb, _SUB)],
                             osem.at[buf])
            return c2
        lax.fori_loop(0, nsub, sub_body, 0)
        for t in (nsub - 2, nsub - 1):
            pltpu.make_async_copy(
                outbuf.at[t % 2],
                out.at[pl.ds(boff + nbase + t * _SUB, _SUB)],
                osem.at[t % 2]).wait()
        return c0
    lax.fori_loop(0, _B, batch_body, 0)


def _sc_call(xlin_flat, ai_flat, aj_flat, tk_flat):
    f = pl.kernel(
        _sc_body,
        out_type=jax.ShapeDtypeStruct((_B * _N, _D), jnp.float32),
        mesh=plsc.VectorSubcoreMesh(core_axis_name="c", subcore_axis_name="s",
                                    num_cores=_NC, num_subcores=_NS),
        compiler_params=pltpu.CompilerParams(needs_layout_passes=False,
                                             use_tc_tiling_on_sc=False),
        scratch_types=[
            pltpu.VMEM((_CHUNK * _KP,), jnp.int32),
            pltpu.VMEM((_CHUNK * _KL,), jnp.int32),
            pltpu.VMEM((_N,), jnp.float32),
            pltpu.VMEM((_CHUNK,), jnp.float32),
            pltpu.VMEM((2, _SUB * _KL, _D), jnp.float32),
            pltpu.VMEM((_SUB, _KP), jnp.float32),
            pltpu.VMEM((2, _SUB, _D), jnp.float32),
            pltpu.SemaphoreType.DMA((2,)),
            pltpu.SemaphoreType.DMA((2,)),
        ],
    )
    return f(xlin_flat, ai_flat, aj_flat, tk_flat)


def _epi_body(agg_ref, emb_ref, glb_ref, g1_ref, b1_ref, g2_ref, b2_ref,
              wo_ref, o_ref):
    a = agg_ref[...] + glb_ref[...]
    h = jnp.maximum(a * g1_ref[...] + b1_ref[...], 0.0)
    o = jnp.maximum(h * emb_ref[...] * g2_ref[...] + b2_ref[...], 0.0)
    o_ref[0, 0] = jnp.sum(o * wo_ref[...], axis=1)


def _epi_call(agg, emb_table, glb, g1, b1, g2, b2, wo):
    vspec = pl.BlockSpec((1, _D), lambda b: (0, 0))
    return pl.pallas_call(
        _epi_body,
        grid=(_B,),
        in_specs=[pl.BlockSpec((_N, _D), lambda b: (b, 0)),
                  pl.BlockSpec((_N, _D), lambda b: (0, 0)),
                  vspec, vspec, vspec, vspec, vspec, vspec],
        out_specs=pl.BlockSpec((1, 1, _N), lambda b: (b, 0, 0)),
        out_shape=jax.ShapeDtypeStruct((_B, 1, _N), jnp.float32),
    )(agg, emb_table, glb, g1, b1, g2, b2, wo)


def kernel(batch_tensor, org_edge_index, emb_table, lin_W, att_i, att_j,
           att_em_i, att_em_j, gl_bias, bn1_gamma, bn1_beta, bn_out_gamma,
           bn_out_beta, out_W, out_b):
    del org_edge_index
    tk32 = _topk_call(emb_table)
    xlin, ai, aj = _lin_call(batch_tensor, lin_W, emb_table,
                             att_i, att_j, att_em_i, att_em_j)
    agg = _sc_call(xlin.reshape(_B * _N, _D), ai.reshape(-1), aj.reshape(-1),
                   tk32.reshape(-1))
    c = (1.0 + _BN_EPS) ** -0.5
    o = _epi_call(agg, emb_table,
                  gl_bias.reshape(1, _D),
                  (bn1_gamma * c).reshape(1, _D), bn1_beta.reshape(1, _D),
                  (bn_out_gamma * c).reshape(1, _D), bn_out_beta.reshape(1, _D),
                  out_W.reshape(1, _D))
    return o.reshape(_B, _N) + out_b[0]
